# baseline (speedup 1.0000x reference)
import functools

import jax
import jax.numpy as jnp
from jax.experimental import pallas as pl
from jax.experimental.pallas import tpu as pltpu


def _proj_body(x_ref, w_ref, asrc_ref, adst_ref, wh_ref, s_ref, d_ref):
    wh = jnp.dot(x_ref[...], w_ref[...], preferred_element_type=jnp.float32)
    wh_ref[...] = wh
    s_ref[...] = jnp.dot(wh, asrc_ref[...], preferred_element_type=jnp.float32)
    d_ref[...] = jnp.dot(wh, adst_ref[...], preferred_element_type=jnp.float32)


def _proj(x, w, a_src, a_dst, blk):
    n, f = x.shape
    h = w.shape[1]
    grid = (pl.cdiv(n, blk),)
    return pl.pallas_call(
        _proj_body,
        grid=grid,
        in_specs=[
            pl.BlockSpec((blk, f), lambda i: (i, 0)),
            pl.BlockSpec((f, h), lambda i: (0, 0)),
            pl.BlockSpec((h, 1), lambda i: (0, 0)),
            pl.BlockSpec((h, 1), lambda i: (0, 0)),
        ],
        out_specs=[
            pl.BlockSpec((blk, h), lambda i: (i, 0)),
            pl.BlockSpec((blk, 1), lambda i: (i, 0)),
            pl.BlockSpec((blk, 1), lambda i: (i, 0)),
        ],
        out_shape=[
            jax.ShapeDtypeStruct((n, h), jnp.float32),
            jax.ShapeDtypeStruct((n, 1), jnp.float32),
            jax.ShapeDtypeStruct((n, 1), jnp.float32),
        ],
    )(x, w, a_src, a_dst)


def _factors_body(s_ref, d_ref, a_ref, c_ref, b_ref, dd_ref):
    s = s_ref[...]
    d = d_ref[...]
    dmax = jnp.max(d)
    t = s + dmax
    m = jnp.where(t > 0, t, 0.2 * t)
    a_ref[...] = jnp.exp(t - m)
    c_ref[...] = jnp.exp(0.2 * t - m)
    b_ref[...] = jnp.exp(d - dmax)
    dd_ref[...] = jnp.exp(0.2 * (d - dmax))


def _factors(s_row, d_row):
    n = s_row.shape[1]
    shp = jax.ShapeDtypeStruct((1, n), jnp.float32)
    return pl.pallas_call(
        _factors_body,
        grid=(1,),
        in_specs=[pl.BlockSpec((1, n), lambda i: (0, 0))] * 2,
        out_specs=[pl.BlockSpec((1, n), lambda i: (0, 0))] * 4,
        out_shape=[shp, shp, shp, shp],
    )(s_row, d_row)


def _epilogue(out_ref, l_ref):
    l = l_ref[...]
    l = jnp.where(l > 0, l, 1.0)
    o = out_ref[...] / l
    out_ref[...] = jnp.where(o > 0, o, jnp.exp(o) - 1.0)


def _flash_pack_body(adj_ref, wh_ref, a_ref, c_ref, b_ref, d_ref,
                     out_ref, msk_ref, flg_ref, l_ref):
    j = pl.program_id(1)
    nj = pl.num_programs(1)

    @pl.when(j == 0)
    def _init():
        out_ref[...] = jnp.zeros_like(out_ref)
        flg_ref[...] = jnp.zeros_like(flg_ref)
        l_ref[...] = jnp.zeros_like(l_ref)

    adj = adj_ref[...]
    keep = adj > 0
    msk_ref[...] = keep.astype(jnp.int8)
    zflag = jnp.max(jnp.where(adj <= 0, 1.0, 0.0))
    lane = jax.lax.broadcasted_iota(jnp.int32, flg_ref.shape, 2)
    flg_ref[...] += jnp.where(lane == j, zflag, 0.0)
    ab = a_ref[...] * b_ref[...]
    cd = c_ref[...] * d_ref[...]
    p = jnp.maximum(ab, cd)
    p = jnp.where(keep, p, 0.0)
    l_ref[...] += jnp.sum(p, axis=1, keepdims=True)
    out_ref[...] += jnp.dot(p, wh_ref[...], preferred_element_type=jnp.float32)

    @pl.when(j == nj - 1)
    def _fini():
        _epilogue(out_ref, l_ref)


def _flash_mask_body(flags_ref, msk_ref, wh_ref, a_ref, c_ref, b_ref, d_ref,
                     out_ref, l_ref):
    j = pl.program_id(1)
    nj = pl.num_programs(1)

    @pl.when(j == 0)
    def _init():
        out_ref[...] = jnp.zeros_like(out_ref)
        l_ref[...] = jnp.zeros_like(l_ref)

    flag = flags_ref[pl.program_id(0) * nj + j]

    @pl.when(flag != 0)
    def _masked():
        ab = a_ref[...] * b_ref[...]
        cd = c_ref[...] * d_ref[...]
        p = jnp.maximum(ab, cd)
        p = jnp.where(msk_ref[...] != 0, p, 0.0)
        l_ref[...] += jnp.sum(p, axis=1, keepdims=True)
        out_ref[...] += jnp.dot(p, wh_ref[...],
                                preferred_element_type=jnp.float32)

    @pl.when(flag == 0)
    def _clean():
        ab = a_ref[...] * b_ref[...]
        cd = c_ref[...] * d_ref[...]
        p = jnp.maximum(ab, cd)
        l_ref[...] += jnp.sum(p, axis=1, keepdims=True)
        out_ref[...] += jnp.dot(p, wh_ref[...],
                                preferred_element_type=jnp.float32)

    @pl.when(j == nj - 1)
    def _fini():
        _epilogue(out_ref, l_ref)


def _flash_layer(adj_or_mask, wh_p, a_col, c_col, b_pad, d_pad, rblk, cblk,
                 flags=None):
    n = adj_or_mask.shape[0]
    h = wh_p.shape[1]
    p_cols = b_pad.shape[1]
    ni, nj = pl.cdiv(n, rblk), p_cols // cblk
    emit_mask = flags is None
    in_specs = [
        pl.BlockSpec((rblk, cblk), lambda i, j: (i, j)),
        pl.BlockSpec((cblk, h), lambda i, j: (j, 0)),
        pl.BlockSpec((rblk, 1), lambda i, j: (i, 0)),
        pl.BlockSpec((rblk, 1), lambda i, j: (i, 0)),
        pl.BlockSpec((1, cblk), lambda i, j: (0, j)),
        pl.BlockSpec((1, cblk), lambda i, j: (0, j)),
    ]
    args = [adj_or_mask, wh_p, a_col, c_col, b_pad, d_pad]
    out_shape = [jax.ShapeDtypeStruct((n, h), jnp.float32)]
    out_specs = [pl.BlockSpec((rblk, h), lambda i, j: (i, 0))]
    if emit_mask:
        out_shape.append(jax.ShapeDtypeStruct((n, p_cols), jnp.int8))
        out_specs.append(pl.BlockSpec((rblk, cblk), lambda i, j: (i, j)))
        out_shape.append(jax.ShapeDtypeStruct((ni, 1, 128), jnp.float32))
        out_specs.append(pl.BlockSpec((1, 1, 128), lambda i, j: (i, 0, 0)))
    else:
        in_specs.insert(0, pl.BlockSpec(memory_space=pltpu.SMEM))
        args.insert(0, flags)
    res = pl.pallas_call(
        _flash_pack_body if emit_mask else _flash_mask_body,
        grid=(ni, nj),
        in_specs=in_specs,
        out_specs=out_specs,
        out_shape=out_shape,
        scratch_shapes=[pltpu.VMEM((rblk, 1), jnp.float32)],
        compiler_params=pltpu.CompilerParams(
            dimension_semantics=("parallel", "arbitrary")),
    )(*args)
    return res if emit_mask else res[0]


def _head_body(x_ref, we_ref, be_ref, out_ref):
    logits = jnp.dot(x_ref[...], we_ref[...], preferred_element_type=jnp.float32)
    logits = logits + be_ref[...]
    m = jnp.max(logits, axis=1, keepdims=True)
    z = logits - m
    lse = jnp.log(jnp.sum(jnp.exp(z), axis=1, keepdims=True))
    out_ref[...] = z - lse


def _head(x, we, be_row, blk):
    n, h = x.shape
    c = we.shape[1]
    return pl.pallas_call(
        _head_body,
        grid=(pl.cdiv(n, blk),),
        in_specs=[
            pl.BlockSpec((blk, h), lambda i: (i, 0)),
            pl.BlockSpec((h, c), lambda i: (0, 0)),
            pl.BlockSpec((1, c), lambda i: (0, 0)),
        ],
        out_specs=pl.BlockSpec((blk, c), lambda i: (i, 0)),
        out_shape=jax.ShapeDtypeStruct((n, c), jnp.float32),
    )(x, we, be_row)


def _gat_layer_fast(x, adj_or_mask, w, a, rblk, cblk, flags=None):
    n = x.shape[0]
    h = w.shape[1]
    a_src = a[:h]
    a_dst = a[h:]
    wh, s, d = _proj(x, w, a_src, a_dst, blk=1000 if n % 1000 == 0 else 8 * pl.cdiv(n, 8))
    av, cv, bv, dv = _factors(s.reshape(1, n), d.reshape(1, n))

    p_cols = cblk * pl.cdiv(n, cblk)
    pad = p_cols - n
    b_pad = jnp.pad(bv, ((0, 0), (0, pad)))
    d_pad = jnp.pad(dv, ((0, 0), (0, pad)))
    wh_p = jnp.pad(wh, ((0, pad), (0, 0)))
    return _flash_layer(adj_or_mask, wh_p, av.reshape(n, 1), cv.reshape(n, 1),
                        b_pad, d_pad, rblk, cblk, flags)


@functools.partial(jax.jit, static_argnames=())
def kernel(feat, adj, W1, a1, W2, a2, W3, a3, We, be):
    n = feat.shape[0]
    rblk, cblk = 1000, 2560
    x, mask, flags_f = _gat_layer_fast(feat, adj, W1, a1, rblk, cblk)
    nj = mask.shape[1] // cblk
    flags = (flags_f[:, 0, :nj] > 0).astype(jnp.int32).reshape(-1)
    x = _gat_layer_fast(x, mask, W2, a2, rblk, cblk, flags=flags)
    x = _gat_layer_fast(x, mask, W3, a3, rblk, cblk, flags=flags)
    return _head(x, We, be.reshape(1, -1), blk=1000 if n % 1000 == 0 else 8 * pl.cdiv(n, 8))


# speedup vs baseline: 1.0699x; 1.0699x over previous
import functools

import jax
import jax.numpy as jnp
from jax.experimental import pallas as pl
from jax.experimental.pallas import tpu as pltpu


def _proj_body(n, blk, x_ref, w_ref, asrc_ref, adst_ref, wh_ref, s_ref, d_ref):
    i = pl.program_id(0)
    wh = jnp.dot(x_ref[...], w_ref[...], preferred_element_type=jnp.float32)
    row = jax.lax.broadcasted_iota(jnp.int32, wh.shape, 0) + i * blk
    wh = jnp.where(row < n, wh, 0.0)
    wh_ref[...] = wh
    s_ref[...] = jnp.dot(wh, asrc_ref[...], preferred_element_type=jnp.float32)
    d_ref[...] = jnp.dot(wh, adst_ref[...], preferred_element_type=jnp.float32)


def _proj(x, w, a_src, a_dst, p_rows, blk):
    n, f = x.shape
    h = w.shape[1]
    grid = (p_rows // blk,)
    return pl.pallas_call(
        functools.partial(_proj_body, n, blk),
        grid=grid,
        in_specs=[
            pl.BlockSpec((blk, f), lambda i: (i, 0)),
            pl.BlockSpec((f, h), lambda i: (0, 0)),
            pl.BlockSpec((h, 1), lambda i: (0, 0)),
            pl.BlockSpec((h, 1), lambda i: (0, 0)),
        ],
        out_specs=[
            pl.BlockSpec((blk, h), lambda i: (i, 0)),
            pl.BlockSpec((blk, 1), lambda i: (i, 0)),
            pl.BlockSpec((blk, 1), lambda i: (i, 0)),
        ],
        out_shape=[
            jax.ShapeDtypeStruct((p_rows, h), jnp.float32),
            jax.ShapeDtypeStruct((p_rows, 1), jnp.float32),
            jax.ShapeDtypeStruct((p_rows, 1), jnp.float32),
        ],
    )(x, w, a_src, a_dst)


def _factors_body(n, s_ref, d_ref, a_ref, c_ref, b_ref, dd_ref):
    s = s_ref[...]
    d = d_ref[...]
    valid = jax.lax.broadcasted_iota(jnp.int32, d.shape, 1) < n
    dmax = jnp.max(jnp.where(valid, d, -3e38))
    t = s + dmax
    m = jnp.where(t > 0, t, 0.2 * t)
    a_ref[...] = jnp.exp(t - m)
    c_ref[...] = jnp.exp(0.2 * t - m)
    b_ref[...] = jnp.where(valid, jnp.exp(d - dmax), 0.0)
    dd_ref[...] = jnp.where(valid, jnp.exp(0.2 * (d - dmax)), 0.0)


def _factors(s_row, d_row, n):
    p = s_row.shape[1]
    shp = jax.ShapeDtypeStruct((1, p), jnp.float32)
    return pl.pallas_call(
        functools.partial(_factors_body, n),
        grid=(1,),
        in_specs=[pl.BlockSpec((1, p), lambda i: (0, 0))] * 2,
        out_specs=[pl.BlockSpec((1, p), lambda i: (0, 0))] * 4,
        out_shape=[shp, shp, shp, shp],
    )(s_row, d_row)


def _epilogue(out_ref, l_ref):
    l = l_ref[...]
    l = jnp.where(l > 0, l, 1.0)
    o = out_ref[...] / l
    out_ref[...] = jnp.where(o > 0, o, jnp.exp(o) - 1.0)


def _flash_pack_body(adj_ref, wh_ref, a_ref, c_ref, b_ref, d_ref,
                     out_ref, msk_ref, flg_ref, l_ref):
    j = pl.program_id(1)
    nj = pl.num_programs(1)

    @pl.when(j == 0)
    def _init():
        out_ref[...] = jnp.zeros_like(out_ref)
        flg_ref[...] = jnp.zeros_like(flg_ref)
        l_ref[...] = jnp.zeros_like(l_ref)

    adj = adj_ref[...]
    keep = adj > 0
    msk_ref[...] = keep.astype(jnp.int8)
    zflag = jnp.max(jnp.where(adj <= 0, 1.0, 0.0))
    lane = jax.lax.broadcasted_iota(jnp.int32, flg_ref.shape, 2)
    flg_ref[...] += jnp.where(lane == j, zflag, 0.0)
    ab = a_ref[...] * b_ref[...]
    cd = c_ref[...] * d_ref[...]
    p = jnp.maximum(ab, cd)
    p = jnp.where(keep, p, 0.0)
    l_ref[...] += jnp.sum(p, axis=1, keepdims=True)
    out_ref[...] += jnp.dot(p, wh_ref[...], preferred_element_type=jnp.float32)

    @pl.when(j == nj - 1)
    def _fini():
        _epilogue(out_ref, l_ref)


def _flash_mask_body(flags_ref, msk_ref, wh_ref, a_ref, c_ref, b_ref, d_ref,
                     out_ref, l_ref):
    j = pl.program_id(1)
    nj = pl.num_programs(1)

    @pl.when(j == 0)
    def _init():
        out_ref[...] = jnp.zeros_like(out_ref)
        l_ref[...] = jnp.zeros_like(l_ref)

    flag = flags_ref[pl.program_id(0) * nj + j]

    @pl.when(flag != 0)
    def _masked():
        ab = a_ref[...] * b_ref[...]
        cd = c_ref[...] * d_ref[...]
        p = jnp.maximum(ab, cd)
        p = jnp.where(msk_ref[...] != 0, p, 0.0)
        l_ref[...] += jnp.sum(p, axis=1, keepdims=True)
        out_ref[...] += jnp.dot(p, wh_ref[...],
                                preferred_element_type=jnp.float32)

    @pl.when(flag == 0)
    def _clean():
        ab = a_ref[...] * b_ref[...]
        cd = c_ref[...] * d_ref[...]
        p = jnp.maximum(ab, cd)
        l_ref[...] += jnp.sum(p, axis=1, keepdims=True)
        out_ref[...] += jnp.dot(p, wh_ref[...],
                                preferred_element_type=jnp.float32)

    @pl.when(j == nj - 1)
    def _fini():
        _epilogue(out_ref, l_ref)


def _flash_mask_head_body(flags_ref, msk_ref, wh_ref, a_ref, c_ref, b_ref,
                          d_ref, we_ref, be_ref, out_ref, acc_ref, l_ref):
    j = pl.program_id(1)
    nj = pl.num_programs(1)

    @pl.when(j == 0)
    def _init():
        acc_ref[...] = jnp.zeros_like(acc_ref)
        l_ref[...] = jnp.zeros_like(l_ref)

    flag = flags_ref[pl.program_id(0) * nj + j]

    @pl.when(flag != 0)
    def _masked():
        ab = a_ref[...] * b_ref[...]
        cd = c_ref[...] * d_ref[...]
        p = jnp.maximum(ab, cd)
        p = jnp.where(msk_ref[...] != 0, p, 0.0)
        l_ref[...] += jnp.sum(p, axis=1, keepdims=True)
        acc_ref[...] += jnp.dot(p, wh_ref[...],
                                preferred_element_type=jnp.float32)

    @pl.when(flag == 0)
    def _clean():
        ab = a_ref[...] * b_ref[...]
        cd = c_ref[...] * d_ref[...]
        p = jnp.maximum(ab, cd)
        l_ref[...] += jnp.sum(p, axis=1, keepdims=True)
        acc_ref[...] += jnp.dot(p, wh_ref[...],
                                preferred_element_type=jnp.float32)

    @pl.when(j == nj - 1)
    def _fini():
        l = l_ref[...]
        l = jnp.where(l > 0, l, 1.0)
        o = acc_ref[...] / l
        o = jnp.where(o > 0, o, jnp.exp(o) - 1.0)
        lg = jnp.dot(o, we_ref[...], preferred_element_type=jnp.float32)
        lg = lg + be_ref[...]
        z = lg - jnp.max(lg, axis=1, keepdims=True)
        out_ref[...] = z - jnp.log(jnp.sum(jnp.exp(z), axis=1, keepdims=True))


def _flash_layer(adj_or_mask, wh_p, a_col, c_col, b_pad, d_pad, rblk, cblk,
                 flags=None, head=None):
    n = adj_or_mask.shape[0]
    h = wh_p.shape[1]
    p_cols = b_pad.shape[1]
    ni, nj = pl.cdiv(n, rblk), p_cols // cblk
    emit_mask = flags is None
    in_specs = [
        pl.BlockSpec((rblk, cblk), lambda i, j: (i, j)),
        pl.BlockSpec((cblk, h), lambda i, j: (j, 0)),
        pl.BlockSpec((rblk, 1), lambda i, j: (i, 0)),
        pl.BlockSpec((rblk, 1), lambda i, j: (i, 0)),
        pl.BlockSpec((1, cblk), lambda i, j: (0, j)),
        pl.BlockSpec((1, cblk), lambda i, j: (0, j)),
    ]
    args = [adj_or_mask, wh_p, a_col, c_col, b_pad, d_pad]
    out_shape = [jax.ShapeDtypeStruct((n, h), jnp.float32)]
    out_specs = [pl.BlockSpec((rblk, h), lambda i, j: (i, 0))]
    if emit_mask:
        out_shape.append(jax.ShapeDtypeStruct((n, p_cols), jnp.int8))
        out_specs.append(pl.BlockSpec((rblk, cblk), lambda i, j: (i, j)))
        out_shape.append(jax.ShapeDtypeStruct((ni, 1, 128), jnp.float32))
        out_specs.append(pl.BlockSpec((1, 1, 128), lambda i, j: (i, 0, 0)))
    else:
        in_specs.insert(0, pl.BlockSpec(memory_space=pltpu.SMEM))
        args.insert(0, flags)
    scratch = [pltpu.VMEM((rblk, 1), jnp.float32)]
    body = _flash_pack_body if emit_mask else _flash_mask_body
    if head is not None:
        we, be_row = head
        nclass = we.shape[1]
        in_specs.extend([
            pl.BlockSpec((we.shape[0], nclass), lambda i, j: (0, 0)),
            pl.BlockSpec((1, nclass), lambda i, j: (0, 0)),
        ])
        args.extend([we, be_row])
        out_shape = [jax.ShapeDtypeStruct((n, nclass), jnp.float32)]
        out_specs = [pl.BlockSpec((rblk, nclass), lambda i, j: (i, 0))]
        scratch.insert(0, pltpu.VMEM((rblk, h), jnp.float32))
        body = _flash_mask_head_body
    res = pl.pallas_call(
        body,
        grid=(ni, nj),
        in_specs=in_specs,
        out_specs=out_specs,
        out_shape=out_shape,
        scratch_shapes=scratch,
        compiler_params=pltpu.CompilerParams(
            dimension_semantics=("parallel", "arbitrary")),
    )(*args)
    return res if emit_mask else res[0]


def _gat_layer_fast(x, adj_or_mask, w, a, rblk, cblk, flags=None, head=None):
    n = adj_or_mask.shape[0]
    h = w.shape[1]
    a_src = a[:h]
    a_dst = a[h:]
    p_cols = cblk * pl.cdiv(n, cblk)
    wh_p, s, d = _proj(x, w, a_src, a_dst, p_cols, blk=1024)
    av, cv, b_pad, d_pad = _factors(s.reshape(1, p_cols), d.reshape(1, p_cols), n)
    return _flash_layer(adj_or_mask, wh_p, av.reshape(p_cols, 1),
                        cv.reshape(p_cols, 1), b_pad, d_pad, rblk, cblk,
                        flags, head)


@functools.partial(jax.jit, static_argnames=())
def kernel(feat, adj, W1, a1, W2, a2, W3, a3, We, be):
    rblk, cblk = 1000, 2048
    x, mask, flags_f = _gat_layer_fast(feat, adj, W1, a1, rblk, cblk)
    nj = mask.shape[1] // cblk
    flags = (flags_f[:, 0, :nj] > 0).astype(jnp.int32).reshape(-1)
    x = _gat_layer_fast(x, mask, W2, a2, rblk, cblk, flags=flags)
    return _gat_layer_fast(x, mask, W3, a3, rblk, cblk, flags=flags,
                           head=(We, be.reshape(1, -1)))
